# baseline (speedup 1.0000x reference)
import math

import jax
import jax.numpy as jnp
from jax.experimental import pallas as pl
from jax.experimental.pallas import tpu as pltpu

_BINS = 9
_CELL = 8
_H = 512
_W = 512
_TAN_PI8 = math.tan(math.pi / 8)


def _hog_body(x_ref, out_ref):
    xs = x_ref[0].astype(jnp.float32)
    s = xs[0] + xs[1] + xs[2]

    zrow = jnp.zeros((1, _W), jnp.float32)
    s_up = jnp.concatenate([s[1:], zrow], axis=0)
    s_dn = jnp.concatenate([zrow, s[:-1]], axis=0)
    a = s_dn + 2.0 * s + s_up
    d = s_up - s_dn

    zcol = jnp.zeros((_H, 1), jnp.float32)
    a_r = jnp.concatenate([a[:, 1:], zcol], axis=1)
    a_l = jnp.concatenate([zcol, a[:, :-1]], axis=1)
    gx = a_r - a_l
    d_r = jnp.concatenate([d[:, 1:], zcol], axis=1)
    d_l = jnp.concatenate([zcol, d[:, :-1]], axis=1)
    gy = d_l + 2.0 * d + d_r

    mag = jnp.sqrt(gx * gx + gy * gy)

    u = jnp.abs(gx)
    w = jnp.abs(gy)
    s1 = gy <= 0.0
    s2 = (u > w) | (~s1 & (u == w))
    mn = jnp.minimum(u, w)
    mx = jnp.maximum(u, w)
    s3 = mn >= mx * _TAN_PI8
    s4 = (u <= 0.0) & (gy < 0.0)
    b2 = s1 ^ s2
    b0 = b2 ^ s3
    binf = (jnp.where(s1, 4.0, 0.0) + jnp.where(b2, 2.0, 0.0)
            + jnp.where(b0, 1.0, 0.0) + jnp.where(s4, 1.0, 0.0))

    mag_bf = mag.astype(jnp.bfloat16)
    bin_bf = binf.astype(jnp.bfloat16)

    ji = jax.lax.broadcasted_iota(jnp.int32, (_W, _W // _CELL), 0)
    ci = jax.lax.broadcasted_iota(jnp.int32, (_W, _W // _CELL), 1)
    pool = jnp.where(ji // _CELL == ci, 1.0, 0.0).astype(jnp.bfloat16)
    ci2 = jax.lax.broadcasted_iota(jnp.int32, (_H // _CELL, _H), 0)
    ii = jax.lax.broadcasted_iota(jnp.int32, (_H // _CELL, _H), 1)
    poolT = jnp.where(ii // _CELL == ci2, 1.0, 0.0).astype(jnp.bfloat16)

    for b in range(_BINS):
        mb = jnp.where(bin_bf == jnp.bfloat16(b), mag_bf, jnp.bfloat16(0))
        cm = jax.lax.dot_general(mb, pool, (((1,), (0,)), ((), ())),
                                 preferred_element_type=jnp.float32)
        rp = jax.lax.dot_general(poolT, cm.astype(jnp.bfloat16),
                                 (((1,), (0,)), ((), ())),
                                 preferred_element_type=jnp.float32)
        out_ref[0, b] = rp


def kernel(x, sobel_x_w, sobel_y_w):
    del sobel_x_w, sobel_y_w
    n = x.shape[0]
    xb = x.astype(jnp.bfloat16)
    out = pl.pallas_call(
        _hog_body,
        grid=(n,),
        in_specs=[pl.BlockSpec((1, 3, _H, _W), lambda i: (i, 0, 0, 0))],
        out_specs=pl.BlockSpec((1, _BINS, _H // _CELL, _W // _CELL),
                               lambda i: (i, 0, 0, 0)),
        out_shape=jax.ShapeDtypeStruct((n, _BINS, _H // _CELL, _W // _CELL),
                                       jnp.float32),
        compiler_params=pltpu.CompilerParams(
            dimension_semantics=("arbitrary",)),
    )(xb)
    return out.reshape(n, -1)


# speedup vs baseline: 1.5672x; 1.5672x over previous
import math

import jax
import jax.numpy as jnp
from jax.experimental import pallas as pl
from jax.experimental.pallas import tpu as pltpu

_BINS = 9
_CELL = 8
_H = 512
_W = 512
_TAN_PI8 = math.tan(math.pi / 8)


def _hog_body(x_ref, out_ref):
    xs = x_ref[0].astype(jnp.bfloat16).astype(jnp.float32)
    s = xs[0] + xs[1] + xs[2]

    zrow = jnp.zeros((1, _W), jnp.float32)
    s_up = jnp.concatenate([s[1:], zrow], axis=0)
    s_dn = jnp.concatenate([zrow, s[:-1]], axis=0)
    a = s_dn + 2.0 * s + s_up
    d = s_up - s_dn

    zcol = jnp.zeros((_H, 1), jnp.float32)
    a_r = jnp.concatenate([a[:, 1:], zcol], axis=1)
    a_l = jnp.concatenate([zcol, a[:, :-1]], axis=1)
    gx = a_r - a_l
    d_r = jnp.concatenate([d[:, 1:], zcol], axis=1)
    d_l = jnp.concatenate([zcol, d[:, :-1]], axis=1)
    gy = d_l + 2.0 * d + d_r

    mag = jnp.sqrt(gx * gx + gy * gy)

    u = jnp.abs(gx)
    w = jnp.abs(gy)
    s1 = gy <= 0.0
    s2 = (u > w) | (~s1 & (u == w))
    mn = jnp.minimum(u, w)
    mx = jnp.maximum(u, w)
    s3 = mn >= mx * _TAN_PI8
    s4 = (u <= 0.0) & (gy < 0.0)
    b2 = s1 ^ s2
    b0 = b2 ^ s3
    binf = (jnp.where(s1, 4.0, 0.0) + jnp.where(b2, 2.0, 0.0)
            + jnp.where(b0, 1.0, 0.0) + jnp.where(s4, 1.0, 0.0))

    mag_bf = mag.astype(jnp.bfloat16)
    bin_bf = binf.astype(jnp.bfloat16)

    ji = jax.lax.broadcasted_iota(jnp.int32, (_W, _W // _CELL), 0)
    ci = jax.lax.broadcasted_iota(jnp.int32, (_W, _W // _CELL), 1)
    pool = jnp.where(ji // _CELL == ci, 1.0, 0.0).astype(jnp.bfloat16)
    ci2 = jax.lax.broadcasted_iota(jnp.int32, (_H // _CELL, _H), 0)
    ii = jax.lax.broadcasted_iota(jnp.int32, (_H // _CELL, _H), 1)
    poolT = jnp.where(ii // _CELL == ci2, 1.0, 0.0).astype(jnp.bfloat16)

    for b in range(_BINS):
        mb = jnp.where(bin_bf == jnp.bfloat16(b), mag_bf, jnp.bfloat16(0))
        hm = jax.lax.dot_general(poolT, mb, (((1,), (0,)), ((), ())),
                                 preferred_element_type=jnp.float32)
        rp = jax.lax.dot_general(hm.astype(jnp.bfloat16), pool,
                                 (((1,), (0,)), ((), ())),
                                 preferred_element_type=jnp.float32)
        out_ref[0, b] = rp


def kernel(x, sobel_x_w, sobel_y_w):
    del sobel_x_w, sobel_y_w
    n = x.shape[0]
    out = pl.pallas_call(
        _hog_body,
        grid=(n,),
        in_specs=[pl.BlockSpec((1, 3, _H, _W), lambda i: (i, 0, 0, 0))],
        out_specs=pl.BlockSpec((1, _BINS, _H // _CELL, _W // _CELL),
                               lambda i: (i, 0, 0, 0)),
        out_shape=jax.ShapeDtypeStruct((n, _BINS, _H // _CELL, _W // _CELL),
                                       jnp.float32),
        compiler_params=pltpu.CompilerParams(
            dimension_semantics=("arbitrary",)),
    )(x)
    return out.reshape(n, -1)


# speedup vs baseline: 1.7497x; 1.1165x over previous
import math

import jax
import jax.numpy as jnp
from jax.experimental import pallas as pl
from jax.experimental.pallas import tpu as pltpu

_BINS = 9
_CELL = 8
_H = 512
_W = 512
_TAN_PI8 = math.tan(math.pi / 8)


def _hog_body(x_ref, out_ref):
    for li in range(x_ref.shape[0]):
        _hog_one(x_ref, out_ref, li)


def _hog_one(x_ref, out_ref, li):
    xs = x_ref[li].astype(jnp.bfloat16).astype(jnp.float32)
    s = xs[0] + xs[1] + xs[2]

    zrow = jnp.zeros((1, _W), jnp.float32)
    s_up = jnp.concatenate([s[1:], zrow], axis=0)
    s_dn = jnp.concatenate([zrow, s[:-1]], axis=0)
    a = s_dn + 2.0 * s + s_up
    d = s_up - s_dn

    zcol = jnp.zeros((_H, 1), jnp.float32)
    a_r = jnp.concatenate([a[:, 1:], zcol], axis=1)
    a_l = jnp.concatenate([zcol, a[:, :-1]], axis=1)
    gx = a_r - a_l
    d_r = jnp.concatenate([d[:, 1:], zcol], axis=1)
    d_l = jnp.concatenate([zcol, d[:, :-1]], axis=1)
    gy = d_l + 2.0 * d + d_r

    mag = jnp.sqrt(gx * gx + gy * gy)

    u = jnp.abs(gx)
    w = jnp.abs(gy)
    s1 = gy <= 0.0
    s2 = (u > w) | (~s1 & (u == w))
    mn = jnp.minimum(u, w)
    mx = jnp.maximum(u, w)
    s3 = mn >= mx * _TAN_PI8
    s4 = (u <= 0.0) & (gy < 0.0)
    b2 = s1 ^ s2
    b0 = b2 ^ s3
    binf = (jnp.where(s1, 4.0, 0.0) + jnp.where(b2, 2.0, 0.0)
            + jnp.where(b0, 1.0, 0.0) + jnp.where(s4, 1.0, 0.0))

    mag_bf = mag.astype(jnp.bfloat16)
    bin_bf = binf.astype(jnp.bfloat16)

    ji = jax.lax.broadcasted_iota(jnp.int32, (_W, _W // _CELL), 0)
    ci = jax.lax.broadcasted_iota(jnp.int32, (_W, _W // _CELL), 1)
    pool = jnp.where(ji // _CELL == ci, 1.0, 0.0).astype(jnp.bfloat16)
    ci2 = jax.lax.broadcasted_iota(jnp.int32, (_H // _CELL, _H), 0)
    ii = jax.lax.broadcasted_iota(jnp.int32, (_H // _CELL, _H), 1)
    poolT = jnp.where(ii // _CELL == ci2, 1.0, 0.0).astype(jnp.bfloat16)

    for b in range(_BINS):
        mb = jnp.where(bin_bf == jnp.bfloat16(b), mag_bf, jnp.bfloat16(0))
        hm = jax.lax.dot_general(poolT, mb, (((1,), (0,)), ((), ())),
                                 preferred_element_type=jnp.float32)
        rp = jax.lax.dot_general(hm.astype(jnp.bfloat16), pool,
                                 (((1,), (0,)), ((), ())),
                                 preferred_element_type=jnp.float32)
        out_ref[li, b] = rp


def kernel(x, sobel_x_w, sobel_y_w):
    del sobel_x_w, sobel_y_w
    n = x.shape[0]
    imgs_per_step = 2
    out = pl.pallas_call(
        _hog_body,
        grid=(n // imgs_per_step,),
        in_specs=[pl.BlockSpec((imgs_per_step, 3, _H, _W),
                               lambda i: (i, 0, 0, 0))],
        out_specs=pl.BlockSpec((imgs_per_step, _BINS, _H // _CELL,
                                _W // _CELL),
                               lambda i: (i, 0, 0, 0)),
        out_shape=jax.ShapeDtypeStruct((n, _BINS, _H // _CELL, _W // _CELL),
                                       jnp.float32),
        compiler_params=pltpu.CompilerParams(
            dimension_semantics=("arbitrary",)),
    )(x)
    return out.reshape(n, -1)


# speedup vs baseline: 1.8141x; 1.0368x over previous
import math

import jax
import jax.numpy as jnp
from jax.experimental import pallas as pl
from jax.experimental.pallas import tpu as pltpu

_BINS = 9
_CELL = 8
_H = 512
_W = 512
_TAN_PI8 = math.tan(math.pi / 8)


def _hog_body(x_ref, out_ref):
    for li in range(x_ref.shape[0]):
        _hog_one(x_ref, out_ref, li)


def _hog_one(x_ref, out_ref, li):
    xs = x_ref[li].astype(jnp.bfloat16).astype(jnp.float32)
    s = xs[0] + xs[1] + xs[2]

    zrow = jnp.zeros((1, _W), jnp.float32)
    s_up = jnp.concatenate([s[1:], zrow], axis=0)
    s_dn = jnp.concatenate([zrow, s[:-1]], axis=0)
    a = s_dn + 2.0 * s + s_up
    d = s_up - s_dn

    zcol = jnp.zeros((_H, 1), jnp.float32)
    a_r = jnp.concatenate([a[:, 1:], zcol], axis=1)
    a_l = jnp.concatenate([zcol, a[:, :-1]], axis=1)
    gx = a_r - a_l
    d_r = jnp.concatenate([d[:, 1:], zcol], axis=1)
    d_l = jnp.concatenate([zcol, d[:, :-1]], axis=1)
    gy = d_l + 2.0 * d + d_r

    mag = jnp.sqrt(gx * gx + gy * gy)

    u = jnp.abs(gx)
    w = jnp.abs(gy)
    s1 = gy <= 0.0
    s2 = (u > w) | (~s1 & (u == w))
    mn = jnp.minimum(u, w)
    mx = jnp.maximum(u, w)
    s3 = mn >= mx * _TAN_PI8
    s4 = (u <= 0.0) & (gy < 0.0)
    b2 = s1 ^ s2
    b0 = b2 ^ s3
    binf = (jnp.where(s1, 4.0, 0.0) + jnp.where(b2, 2.0, 0.0)
            + jnp.where(b0, 1.0, 0.0) + jnp.where(s4, 1.0, 0.0))

    mag_bf = mag.astype(jnp.bfloat16)
    bin_bf = binf.astype(jnp.bfloat16)

    ji = jax.lax.broadcasted_iota(jnp.int32, (_W, _W // _CELL), 0)
    ci = jax.lax.broadcasted_iota(jnp.int32, (_W, _W // _CELL), 1)
    pool = jnp.where(ji // _CELL == ci, 1.0, 0.0).astype(jnp.bfloat16)
    ci2 = jax.lax.broadcasted_iota(jnp.int32, (_H // _CELL, _H), 0)
    ii = jax.lax.broadcasted_iota(jnp.int32, (_H // _CELL, _H), 1)
    poolT = jnp.where(ii // _CELL == ci2, 1.0, 0.0).astype(jnp.bfloat16)

    for b in range(_BINS):
        mb = jnp.where(bin_bf == jnp.bfloat16(b), mag_bf, jnp.bfloat16(0))
        hm = jax.lax.dot_general(poolT, mb, (((1,), (0,)), ((), ())),
                                 preferred_element_type=jnp.float32)
        rp = jax.lax.dot_general(hm.astype(jnp.bfloat16), pool,
                                 (((1,), (0,)), ((), ())),
                                 preferred_element_type=jnp.float32)
        out_ref[li, b] = rp


def kernel(x, sobel_x_w, sobel_y_w):
    del sobel_x_w, sobel_y_w
    n = x.shape[0]
    imgs_per_step = 4
    out = pl.pallas_call(
        _hog_body,
        grid=(n // imgs_per_step,),
        in_specs=[pl.BlockSpec((imgs_per_step, 3, _H, _W),
                               lambda i: (i, 0, 0, 0))],
        out_specs=pl.BlockSpec((imgs_per_step, _BINS, _H // _CELL,
                                _W // _CELL),
                               lambda i: (i, 0, 0, 0)),
        out_shape=jax.ShapeDtypeStruct((n, _BINS, _H // _CELL, _W // _CELL),
                                       jnp.float32),
        compiler_params=pltpu.CompilerParams(
            dimension_semantics=("arbitrary",)),
    )(x)
    return out.reshape(n, -1)


# speedup vs baseline: 1.9700x; 1.0859x over previous
import math

import jax
import jax.numpy as jnp
from jax.experimental import pallas as pl
from jax.experimental.pallas import tpu as pltpu

_BINS = 9
_CELL = 8
_H = 512
_W = 512
_TAN_PI8 = math.tan(math.pi / 8)


def _hog_body(x_ref, out_ref):
    for li in range(x_ref.shape[0]):
        _hog_one(x_ref, out_ref, li)


def _hog_one(x_ref, out_ref, li):
    xs = x_ref[li].astype(jnp.bfloat16).astype(jnp.float32)
    s = xs[0] + xs[1] + xs[2]

    zrow = jnp.zeros((1, _W), jnp.float32)
    s_up = jnp.concatenate([s[1:], zrow], axis=0)
    s_dn = jnp.concatenate([zrow, s[:-1]], axis=0)
    a = s_dn + 2.0 * s + s_up
    d = s_up - s_dn

    zcol = jnp.zeros((_H, 1), jnp.float32)
    a_r = jnp.concatenate([a[:, 1:], zcol], axis=1)
    a_l = jnp.concatenate([zcol, a[:, :-1]], axis=1)
    gx = a_r - a_l
    d_r = jnp.concatenate([d[:, 1:], zcol], axis=1)
    d_l = jnp.concatenate([zcol, d[:, :-1]], axis=1)
    gy = d_l + 2.0 * d + d_r

    mag = jnp.sqrt(gx * gx + gy * gy)

    u = jnp.abs(gx)
    w = jnp.abs(gy)
    s1 = gy <= 0.0
    s2 = (u > w) | (~s1 & (u == w))
    mn = jnp.minimum(u, w)
    mx = jnp.maximum(u, w)
    s3 = mn >= mx * _TAN_PI8
    s4 = (u <= 0.0) & (gy < 0.0)
    b2 = s1 ^ s2
    b0 = b2 ^ s3
    binf = (jnp.where(s1, 4.0, 0.0) + jnp.where(b2, 2.0, 0.0)
            + jnp.where(b0, 1.0, 0.0) + jnp.where(s4, 1.0, 0.0))

    mag_bf = mag.astype(jnp.bfloat16)
    bin_bf = binf.astype(jnp.bfloat16)

    ji = jax.lax.broadcasted_iota(jnp.int32, (_W, _W // _CELL), 0)
    ci = jax.lax.broadcasted_iota(jnp.int32, (_W, _W // _CELL), 1)
    pool = jnp.where(ji // _CELL == ci, 1.0, 0.0).astype(jnp.bfloat16)
    ci2 = jax.lax.broadcasted_iota(jnp.int32, (_H // _CELL, _H), 0)
    ii = jax.lax.broadcasted_iota(jnp.int32, (_H // _CELL, _H), 1)
    poolT = jnp.where(ii // _CELL == ci2, 1.0, 0.0).astype(jnp.bfloat16)

    hms = []
    for b in range(_BINS):
        mb = jnp.where(bin_bf == jnp.bfloat16(b), mag_bf, jnp.bfloat16(0))
        hm = jax.lax.dot_general(poolT, mb, (((1,), (0,)), ((), ())),
                                 preferred_element_type=jnp.float32)
        hms.append(hm.astype(jnp.bfloat16))
    hm_all = jnp.concatenate(hms, axis=0)
    rp_all = jax.lax.dot_general(hm_all, pool, (((1,), (0,)), ((), ())),
                                 preferred_element_type=jnp.float32)
    out_ref[li] = rp_all.reshape(_BINS, _H // _CELL, _W // _CELL)


def kernel(x, sobel_x_w, sobel_y_w):
    del sobel_x_w, sobel_y_w
    n = x.shape[0]
    imgs_per_step = 4
    out = pl.pallas_call(
        _hog_body,
        grid=(n // imgs_per_step,),
        in_specs=[pl.BlockSpec((imgs_per_step, 3, _H, _W),
                               lambda i: (i, 0, 0, 0))],
        out_specs=pl.BlockSpec((imgs_per_step, _BINS, _H // _CELL,
                                _W // _CELL),
                               lambda i: (i, 0, 0, 0)),
        out_shape=jax.ShapeDtypeStruct((n, _BINS, _H // _CELL, _W // _CELL),
                                       jnp.float32),
        compiler_params=pltpu.CompilerParams(
            dimension_semantics=("arbitrary",)),
    )(x)
    return out.reshape(n, -1)


# speedup vs baseline: 1.9733x; 1.0017x over previous
import math

import jax
import jax.numpy as jnp
from jax.experimental import pallas as pl
from jax.experimental.pallas import tpu as pltpu

_BINS = 9
_CELL = 8
_H = 512
_W = 512
_TAN_PI8 = math.tan(math.pi / 8)


def _hog_body(x_ref, out_ref):
    for li in range(x_ref.shape[0]):
        _hog_one(x_ref, out_ref, li)


def _hog_one(x_ref, out_ref, li):
    xs = x_ref[li].astype(jnp.bfloat16).astype(jnp.float32)
    s = xs[0] + xs[1] + xs[2]

    zrow = jnp.zeros((1, _W), jnp.float32)
    s_up = jnp.concatenate([s[1:], zrow], axis=0)
    s_dn = jnp.concatenate([zrow, s[:-1]], axis=0)
    a = s_dn + 2.0 * s + s_up
    d = s_up - s_dn

    zcol = jnp.zeros((_H, 1), jnp.float32)
    a_r = jnp.concatenate([a[:, 1:], zcol], axis=1)
    a_l = jnp.concatenate([zcol, a[:, :-1]], axis=1)
    gx = a_r - a_l
    d_r = jnp.concatenate([d[:, 1:], zcol], axis=1)
    d_l = jnp.concatenate([zcol, d[:, :-1]], axis=1)
    gy = d_l + 2.0 * d + d_r

    mag = jnp.sqrt(gx * gx + gy * gy)

    u = jnp.abs(gx)
    w = jnp.abs(gy)
    s1 = gy <= 0.0
    s2 = (u > w) | (~s1 & (u == w))
    mn = jnp.minimum(u, w)
    mx = jnp.maximum(u, w)
    s3 = mn >= mx * _TAN_PI8
    s4 = (u <= 0.0) & (gy < 0.0)
    b2 = s1 ^ s2
    b0 = b2 ^ s3
    binf = (jnp.where(s1, 4.0, 0.0) + jnp.where(b2, 2.0, 0.0)
            + jnp.where(b0, 1.0, 0.0) + jnp.where(s4, 1.0, 0.0))

    mag_bf = mag.astype(jnp.bfloat16)
    bin_bf = binf.astype(jnp.bfloat16)

    ji = jax.lax.broadcasted_iota(jnp.int32, (_W, _W // _CELL), 0)
    ci = jax.lax.broadcasted_iota(jnp.int32, (_W, _W // _CELL), 1)
    pool = jnp.where(ji // _CELL == ci, 1.0, 0.0).astype(jnp.bfloat16)
    ci2 = jax.lax.broadcasted_iota(jnp.int32, (_H // _CELL, _H), 0)
    ii = jax.lax.broadcasted_iota(jnp.int32, (_H // _CELL, _H), 1)
    poolT = jnp.where(ii // _CELL == ci2, 1.0, 0.0).astype(jnp.bfloat16)

    hms = []
    for b in range(_BINS):
        mb = jnp.where(bin_bf == jnp.bfloat16(b), mag_bf, jnp.bfloat16(0))
        hm = jax.lax.dot_general(poolT, mb, (((1,), (0,)), ((), ())),
                                 preferred_element_type=jnp.float32)
        hms.append(hm.astype(jnp.bfloat16))
    hm_all = jnp.concatenate(hms, axis=0)
    rp_all = jax.lax.dot_general(hm_all, pool, (((1,), (0,)), ((), ())),
                                 preferred_element_type=jnp.float32)
    out_ref[li] = rp_all.reshape(_BINS, _H // _CELL, _W // _CELL)


def kernel(x, sobel_x_w, sobel_y_w):
    del sobel_x_w, sobel_y_w
    n = x.shape[0]
    imgs_per_step = 4
    out = pl.pallas_call(
        _hog_body,
        grid=(n // imgs_per_step,),
        in_specs=[pl.BlockSpec((imgs_per_step, 3, _H, _W),
                               lambda i: (i, 0, 0, 0))],
        out_specs=pl.BlockSpec((imgs_per_step, _BINS, _H // _CELL,
                                _W // _CELL),
                               lambda i: (i, 0, 0, 0)),
        out_shape=jax.ShapeDtypeStruct((n, _BINS, _H // _CELL, _W // _CELL),
                                       jnp.float32),
        compiler_params=pltpu.CompilerParams(
            dimension_semantics=("parallel",)),
    )(x)
    return out.reshape(n, -1)
